# baseline (speedup 1.0000x reference)
import jax
import jax.numpy as jnp
from jax import lax
from jax.experimental import pallas as pl
from jax.experimental.pallas import tpu as pltpu
from jax.experimental.pallas import tpu_sc as plsc

B = 16384
D = 16

_info = plsc.get_sparse_core_info()
_NC, _NS = _info.num_cores, _info.num_subcores
NW = _NC * _NS
BPW = B // NW

_W = 131072
_NWIN = 8
_TPW = _W // 128
_NT = _NWIN * _TPW
_S_ROWS = _NT * 8
_DP = D // 2


def _pack_pair(lo_f32, hi_f32):
    lo = lax.bitcast_convert_type(lo_f32, jnp.uint32)
    hi = lax.bitcast_convert_type(hi_f32, jnp.uint32)
    word = ((lo + jnp.uint32(0x8000)) >> 16) | ((hi + jnp.uint32(0x8000)) & jnp.uint32(0xFFFF0000))
    return lax.bitcast_convert_type(word, jnp.float32)


def _repack_body(a, oa):
    x = a[...]
    packed = _pack_pair(x[:8, :], x[8:, :])
    y = packed.reshape(8, _TPW, 128)
    oa[...] = y.transpose(1, 0, 2).reshape(_TPW * 8, 128)


def _tc_repack(ta):
    in_spec = pl.BlockSpec((D, _W), lambda w: (0, w))
    out_spec = pl.BlockSpec((_TPW * 8, 128), lambda w: (w, 0))
    s = jax.ShapeDtypeStruct((_S_ROWS, 128), jnp.float32)
    return pl.pallas_call(
        _repack_body,
        grid=(_NWIN,),
        in_specs=[in_spec],
        out_specs=out_spec,
        out_shape=s,
    )(ta)


def _sc_gather_body(ids_hbm, ta,
                    a_o,
                    idx, f, a_v, sem):
    wid = lax.axis_index("s") * _NC + lax.axis_index("c")
    base = wid * BPW
    pltpu.sync_copy(ids_hbm.at[pl.ds(base, BPW)], idx)
    for c in range(BPW // 16):
        sl = pl.ds(c * 16, 16)
        v = idx[sl]
        g = ((v >> 7) << 10) | (v & 127)
        for m in range(_DP):
            f[m, sl] = g + m * 128
    rounds = []
    for m in range(_DP):
        rounds.append(
            pltpu.async_copy(ta.at[f.at[m]], a_v.at[m], sem))
        if m >= 6:
            rounds[m - 6].wait()
    for cp in rounds[-6:]:
        cp.wait()
    pltpu.sync_copy(a_v, a_o.at[:, pl.ds(base, BPW)])


_rowP = jax.ShapeDtypeStruct((_DP, B), jnp.float32)
_sc_gather = pl.kernel(
    _sc_gather_body,
    out_type=_rowP,
    mesh=plsc.VectorSubcoreMesh(core_axis_name="c", subcore_axis_name="s"),
    scratch_types=[
        pltpu.VMEM((BPW,), jnp.int32),
        pltpu.VMEM((_DP, BPW), jnp.int32),
        pltpu.VMEM((_DP, BPW), jnp.float32),
        pltpu.SemaphoreType.DMA,
    ],
    compiler_params=pltpu.CompilerParams(use_tc_tiling_on_sc=False),
)


def _unpack16(x_pk):
    w = lax.bitcast_convert_type(x_pk, jnp.uint32)
    lo = lax.bitcast_convert_type(w << jnp.uint32(16), jnp.float32)
    hi = lax.bitcast_convert_type(w & jnp.uint32(0xFFFF0000), jnp.float32)
    return jnp.concatenate([lo, hi], axis=0)


def _tc_dense_body(gu, gi, mu, mi, w1ta, w1tb, b1, w2t, b2, wpg, wph, bp,
                   out):
    gu16 = _unpack16(gu[...])
    gi16 = _unpack16(gi[...])
    mu16 = _unpack16(mu[...])
    mi16 = _unpack16(mi[...])
    prod = gu16 * gi16
    h1 = jnp.maximum(
        jnp.dot(w1ta[...], mu16, preferred_element_type=jnp.float32)
        + jnp.dot(w1tb[...], mi16, preferred_element_type=jnp.float32)
        + b1[...], 0.0)
    h2 = jnp.maximum(
        jnp.dot(w2t[...], h1, preferred_element_type=jnp.float32) + b2[...],
        0.0)
    r = (jnp.dot(wpg[...], prod, preferred_element_type=jnp.float32)
         + jnp.dot(wph[...], h2, preferred_element_type=jnp.float32)
         + bp[0, 0])
    out[...] = r


_TC_BLK = 2048
_TC_GRID = B // _TC_BLK


def _tc_dense(gu, gi, mu, mi, w1ta, w1tb, b1, w2t, b2, wpg, wph, bp):
    row_spec = pl.BlockSpec((_DP, _TC_BLK), lambda i: (0, i))

    def rep(shape):
        return pl.BlockSpec(shape, lambda i: (0,) * len(shape))

    return pl.pallas_call(
        _tc_dense_body,
        grid=(_TC_GRID,),
        in_specs=[
            row_spec, row_spec, row_spec, row_spec,
            rep((16, D)), rep((16, D)), rep((16, 1)),
            rep((8, 16)), rep((8, 1)),
            rep((1, D)), rep((1, 8)), rep((1, 1)),
        ],
        out_specs=pl.BlockSpec((1, _TC_BLK), lambda i: (0, i)),
        out_shape=jax.ShapeDtypeStruct((1, B), jnp.float32),
    )(gu, gi, mu, mi, w1ta, w1tb, b1, w2t, b2, wpg, wph, bp)


def kernel(U_ids, I_ids, gmf_user_emb, gmf_item_emb, mlp_user_emb,
           mlp_item_emb, W1, b1, W2, b2, Wp, bp):
    u = U_ids.astype(jnp.int32)
    i = I_ids.astype(jnp.int32)
    gu = _sc_gather(u, _tc_repack(gmf_user_emb.T).reshape(-1))
    mu = _sc_gather(u, _tc_repack(mlp_user_emb.T).reshape(-1))
    gi = _sc_gather(i, _tc_repack(gmf_item_emb.T).reshape(-1))
    mi = _sc_gather(i, _tc_repack(mlp_item_emb.T).reshape(-1))
    w1t = W1.T
    r = _tc_dense(gu, gi, mu, mi,
                  w1t[:, :D], w1t[:, D:], b1.reshape(-1, 1),
                  W2.T, b2.reshape(-1, 1),
                  Wp[:D].reshape(1, D), Wp[D:].reshape(1, 8),
                  bp.reshape(1, 1))
    return r.reshape(-1)


# speedup vs baseline: 1.0412x; 1.0412x over previous
import jax
import jax.numpy as jnp
from jax import lax
from jax.experimental import pallas as pl
from jax.experimental.pallas import tpu as pltpu
from jax.experimental.pallas import tpu_sc as plsc

B = 16384
D = 16

_info = plsc.get_sparse_core_info()
_NC, _NS = _info.num_cores, _info.num_subcores
NW = _NC * _NS
BPW = B // NW

_W = 131072
_NWIN = 8
_TPW = _W // 128
_NT = _NWIN * _TPW
_S_ROWS = _NT * 8
_DP = D // 2


def _pack_pair(lo_f32, hi_f32):
    lo = lax.bitcast_convert_type(lo_f32, jnp.uint32)
    hi = lax.bitcast_convert_type(hi_f32, jnp.uint32)
    word = ((lo + jnp.uint32(0x8000)) >> 16) | ((hi + jnp.uint32(0x8000)) & jnp.uint32(0xFFFF0000))
    return lax.bitcast_convert_type(word, jnp.float32)


def _repack_body(a, b, oa, ob):
    for src, dst in ((a, oa), (b, ob)):
        x = src[...]
        packed = _pack_pair(x[:8, :], x[8:, :])
        y = packed.reshape(8, _TPW, 128)
        dst[...] = y.transpose(1, 0, 2).reshape(_TPW * 8, 128)


def _tc_repack(ta, tb):
    in_spec = pl.BlockSpec((D, _W), lambda w: (0, w))
    out_spec = pl.BlockSpec((_TPW * 8, 128), lambda w: (w, 0))
    s = jax.ShapeDtypeStruct((_S_ROWS, 128), jnp.float32)
    return pl.pallas_call(
        _repack_body,
        grid=(_NWIN,),
        in_specs=[in_spec] * 2,
        out_specs=[out_spec] * 2,
        out_shape=[s] * 2,
    )(ta, tb)


def _sc_gather_body(ids_hbm, ta, tb,
                    a_o, b_o,
                    idx, f, a_v, b_v, sem):
    wid = lax.axis_index("s") * _NC + lax.axis_index("c")
    base = wid * BPW
    pltpu.sync_copy(ids_hbm.at[pl.ds(base, BPW)], idx)
    for c in range(BPW // 16):
        sl = pl.ds(c * 16, 16)
        v = idx[sl]
        g = ((v >> 7) << 10) | (v & 127)
        for m in range(_DP):
            f[m, sl] = g + m * 128
    rounds = []
    for m in range(_DP):
        rounds.append([
            pltpu.async_copy(ta.at[f.at[m]], a_v.at[m], sem),
            pltpu.async_copy(tb.at[f.at[m]], b_v.at[m], sem),
        ])
        if m >= 6:
            for cp in rounds[m - 6]:
                cp.wait()
    for r in rounds[-6:]:
        for cp in r:
            cp.wait()
    pltpu.sync_copy(a_v, a_o.at[:, pl.ds(base, BPW)])
    pltpu.sync_copy(b_v, b_o.at[:, pl.ds(base, BPW)])


_rowP = jax.ShapeDtypeStruct((_DP, B), jnp.float32)
_sc_gather = pl.kernel(
    _sc_gather_body,
    out_type=(_rowP, _rowP),
    mesh=plsc.VectorSubcoreMesh(core_axis_name="c", subcore_axis_name="s"),
    scratch_types=[
        pltpu.VMEM((BPW,), jnp.int32),
        pltpu.VMEM((_DP, BPW), jnp.int32),
        pltpu.VMEM((_DP, BPW), jnp.float32),
        pltpu.VMEM((_DP, BPW), jnp.float32),
        pltpu.SemaphoreType.DMA,
    ],
    compiler_params=pltpu.CompilerParams(use_tc_tiling_on_sc=False),
)


def _unpack16(x_pk):
    w = lax.bitcast_convert_type(x_pk, jnp.uint32)
    lo = lax.bitcast_convert_type(w << jnp.uint32(16), jnp.float32)
    hi = lax.bitcast_convert_type(w & jnp.uint32(0xFFFF0000), jnp.float32)
    return jnp.concatenate([lo, hi], axis=0)


def _tc_dense_body(gu, gi, mu, mi, w1ta, w1tb, b1, w2t, b2, wpg, wph, bp,
                   out):
    gu16 = _unpack16(gu[...])
    gi16 = _unpack16(gi[...])
    mu16 = _unpack16(mu[...])
    mi16 = _unpack16(mi[...])
    prod = gu16 * gi16
    h1 = jnp.maximum(
        jnp.dot(w1ta[...], mu16, preferred_element_type=jnp.float32)
        + jnp.dot(w1tb[...], mi16, preferred_element_type=jnp.float32)
        + b1[...], 0.0)
    h2 = jnp.maximum(
        jnp.dot(w2t[...], h1, preferred_element_type=jnp.float32) + b2[...],
        0.0)
    r = (jnp.dot(wpg[...], prod, preferred_element_type=jnp.float32)
         + jnp.dot(wph[...], h2, preferred_element_type=jnp.float32)
         + bp[0, 0])
    out[...] = r


_TC_BLK = 2048
_TC_GRID = B // _TC_BLK


def _tc_dense(gu, gi, mu, mi, w1ta, w1tb, b1, w2t, b2, wpg, wph, bp):
    row_spec = pl.BlockSpec((_DP, _TC_BLK), lambda i: (0, i))

    def rep(shape):
        return pl.BlockSpec(shape, lambda i: (0,) * len(shape))

    return pl.pallas_call(
        _tc_dense_body,
        grid=(_TC_GRID,),
        in_specs=[
            row_spec, row_spec, row_spec, row_spec,
            rep((16, D)), rep((16, D)), rep((16, 1)),
            rep((8, 16)), rep((8, 1)),
            rep((1, D)), rep((1, 8)), rep((1, 1)),
        ],
        out_specs=pl.BlockSpec((1, _TC_BLK), lambda i: (0, i)),
        out_shape=jax.ShapeDtypeStruct((1, B), jnp.float32),
    )(gu, gi, mu, mi, w1ta, w1tb, b1, w2t, b2, wpg, wph, bp)


def kernel(U_ids, I_ids, gmf_user_emb, gmf_item_emb, mlp_user_emb,
           mlp_item_emb, W1, b1, W2, b2, Wp, bp):
    u = U_ids.astype(jnp.int32)
    i = I_ids.astype(jnp.int32)
    s_gu, s_mu = (s.reshape(-1) for s in
                  _tc_repack(gmf_user_emb.T, mlp_user_emb.T))
    gu, mu = _sc_gather(u, s_gu, s_mu)
    s_gi, s_mi = (s.reshape(-1) for s in
                  _tc_repack(gmf_item_emb.T, mlp_item_emb.T))
    gi, mi = _sc_gather(i, s_gi, s_mi)
    w1t = W1.T
    r = _tc_dense(gu, gi, mu, mi,
                  w1t[:, :D], w1t[:, D:], b1.reshape(-1, 1),
                  W2.T, b2.reshape(-1, 1),
                  Wp[:D].reshape(1, D), Wp[D:].reshape(1, 8),
                  bp.reshape(1, 1))
    return r.reshape(-1)
